# baseline (speedup 1.0000x reference)
import functools

import jax
import jax.numpy as jnp
from jax import lax
from jax.experimental import pallas as pl
from jax.experimental.pallas import tpu as pltpu
from jax.experimental.pallas import tpu_sc as plsc

_N = 10000
_E = 160000
_D = 128
_DE = 16
_DU = 16
_B = 64
_NSLOT = 8
_NCONV = 4

_NP = 10112
_NT = _NP // 128
_EP = 163840
_EROWS = _EP // 128
_WORKERS = 32
_CHUNKS = _EROWS // _WORKERS
_NH = 5120
_NAGG = _NH + 128
_ETILE = 1024
_ET = _EP // _ETILE

_f32 = jnp.float32


def _mm(a, b):
    return lax.dot_general(a, b, (((1,), (0,)), ((), ())),
                           preferred_element_type=_f32,
                           precision=lax.Precision.HIGHEST)


def _mmd(a, b):
    return lax.dot_general(a, b, (((1,), (0,)), ((), ())),
                           preferred_element_type=_f32)


def _node_lin_body(x_ref, w_ref, b_ref, o_ref):
    o_ref[...] = _mmd(x_ref[...], w_ref[...]) + b_ref[...]


def _node_lin(x, W, b):
    return pl.pallas_call(
        _node_lin_body,
        grid=(_NT,),
        in_specs=[
            pl.BlockSpec((128, _D), lambda j: (j, 0)),
            pl.BlockSpec((_D, _D), lambda j: (0, 0)),
            pl.BlockSpec((1, _D), lambda j: (0, 0)),
        ],
        out_specs=pl.BlockSpec((128, _D), lambda j: (j, 0)),
        out_shape=jax.ShapeDtypeStruct((_NP, _D), _f32),
    )(x, W, b)


def _pre_body(x_ref, wa_ref, wb_ref, xs_ref, xdu_ref):
    x = x_ref[...]
    xs_ref[...] = _mmd(x, wa_ref[...])
    xdu_ref[...] = _mmd(x, wb_ref[...])


def _pre(x, Wa, Wb):
    full = jax.ShapeDtypeStruct((_NP, _D), _f32)
    return pl.pallas_call(
        _pre_body,
        grid=(_NT,),
        in_specs=[
            pl.BlockSpec((128, _D), lambda j: (j, 0)),
            pl.BlockSpec((_D, _D), lambda j: (0, 0)),
            pl.BlockSpec((_D, _D), lambda j: (0, 0)),
        ],
        out_specs=[pl.BlockSpec((128, _D), lambda j: (j, 0))] * 2,
        out_shape=[full, full],
    )(x, Wa, Wb)


def _eterm_body(ea_ref, slot_ref, bdst_ref, u_ref, w3_ref, st_ref, bm_ref,
                o_ref):
    ea = ea_ref[...]
    ohb = (bdst_ref[...] == lax.broadcasted_iota(jnp.int32, (_ETILE, _B), 1))
    ub = _mm(ohb.astype(_f32), u_ref[...])
    feat3 = jnp.concatenate([ea, ub], axis=1)
    oh = (slot_ref[...] == lax.broadcasted_iota(jnp.int32, (_ETILE, _NSLOT), 1))
    et = _mmd(feat3, w3_ref[0]) + bm_ref[0]
    et = et + _mm(oh.astype(_f32), st_ref[0])
    o_ref[...] = et[None]


def _eterm(ea, slot2d, bdst2d, u, W3_all, slot_tbl, bm_all):
    return pl.pallas_call(
        _eterm_body,
        grid=(_NCONV, _ET),
        in_specs=[
            pl.BlockSpec((_ETILE, _DE), lambda i, j: (j, 0)),
            pl.BlockSpec((_ETILE, 1), lambda i, j: (j, 0)),
            pl.BlockSpec((_ETILE, 1), lambda i, j: (j, 0)),
            pl.BlockSpec((_B, _DU), lambda i, j: (0, 0)),
            pl.BlockSpec((1, _DE + _DU, _D), lambda i, j: (i, 0, 0)),
            pl.BlockSpec((1, _NSLOT, _D), lambda i, j: (i, 0, 0)),
            pl.BlockSpec((1, 1, _D), lambda i, j: (i, 0, 0)),
        ],
        out_specs=pl.BlockSpec((1, _ETILE, _D), lambda i, j: (i, j, 0)),
        out_shape=jax.ShapeDtypeStruct((_NCONV, _EP, _D), _f32),
    )(ea, slot2d, bdst2d, u, W3_all, slot_tbl, bm_all[:, None, :])


def _post_body(with_res, *refs):
    if with_res:
        (x_ref, a0_ref, a1_ref, wu1_ref, wu2_ref, bu_ref, res_ref,
         o_ref) = refs
    else:
        x_ref, a0_ref, a1_ref, wu1_ref, wu2_ref, bu_ref, o_ref = refs
    agg = a0_ref[0] + a1_ref[0]
    z = (_mmd(x_ref[...], wu1_ref[...]) + _mmd(agg, wu2_ref[...])
         + bu_ref[...])
    if with_res:
        z = z + res_ref[...]
    o_ref[...] = jnp.maximum(z, 0.0)


def _post(x, agg_out, Wu1, Wu2, bu, res=None):
    with_res = res is not None
    nt = _NH // 128
    in_specs = [
        pl.BlockSpec((128, _D), lambda j: (j, 0)),
        pl.BlockSpec((1, 128, _D), lambda j: (j // nt, j % nt, 0)),
        pl.BlockSpec((1, 128, _D), lambda j: (j // nt, j % nt, 0)),
        pl.BlockSpec((_D, _D), lambda j: (0, 0)),
        pl.BlockSpec((_D, _D), lambda j: (0, 0)),
        pl.BlockSpec((1, _D), lambda j: (0, 0)),
    ]
    args = [x, agg_out[0], agg_out[1], Wu1, Wu2, bu]
    if with_res:
        in_specs.append(pl.BlockSpec((128, _D), lambda j: (j, 0)))
        args.append(res)
    return pl.pallas_call(
        functools.partial(_post_body, with_res),
        grid=(_NT,),
        in_specs=in_specs,
        out_specs=pl.BlockSpec((128, _D), lambda j: (j, 0)),
        out_shape=jax.ShapeDtypeStruct((_NP, _D), _f32),
    )(*args)


def _pass1_body(x_ref, bat_ref, w1_ref, b1_ref, w2_ref, b2_ref, w3_ref, b3_ref,
                gate_ref, gmax_ref):
    j = pl.program_id(0)
    x = x_ref[...]
    h = jnp.maximum(_mmd(x, w1_ref[...]) + b1_ref[...], 0.0)
    h = jnp.maximum(_mmd(h, w2_ref[...]) + b2_ref[...], 0.0)
    g = _mmd(h, w3_ref[...]) + b3_ref[...]
    gate_ref[...] = g
    oh = (bat_ref[...] == lax.broadcasted_iota(jnp.int32, (128, _B), 1))
    contrib = jnp.where(oh, g, -1e30)
    cmax = jnp.max(contrib, axis=0, keepdims=True)

    @pl.when(j == 0)
    def _():
        gmax_ref[...] = jnp.full((8, _B), -1e30, _f32)

    gmax_ref[...] = jnp.maximum(gmax_ref[...], jnp.broadcast_to(cmax, (8, _B)))


def _pass1(x, batch2d, W1, b1, W2, b2, W3, b3):
    return pl.pallas_call(
        _pass1_body,
        grid=(_NT,),
        in_specs=[
            pl.BlockSpec((128, _D), lambda j: (j, 0)),
            pl.BlockSpec((128, 1), lambda j: (j, 0)),
            pl.BlockSpec((_D, 128), lambda j: (0, 0)),
            pl.BlockSpec((1, 128), lambda j: (0, 0)),
            pl.BlockSpec((128, 128), lambda j: (0, 0)),
            pl.BlockSpec((1, 128), lambda j: (0, 0)),
            pl.BlockSpec((128, 1), lambda j: (0, 0)),
            pl.BlockSpec((1, 1), lambda j: (0, 0)),
        ],
        out_specs=[
            pl.BlockSpec((128, 1), lambda j: (j, 0)),
            pl.BlockSpec((8, _B), lambda j: (0, 0)),
        ],
        out_shape=[
            jax.ShapeDtypeStruct((_NP, 1), _f32),
            jax.ShapeDtypeStruct((8, _B), _f32),
        ],
    )(x, batch2d, W1, b1, W2, b2, W3, b3)


def _pass2_body(x_ref, gate_ref, gmax_ref, bat_ref,
                w1_ref, b1_ref, w2_ref, b2_ref, w3_ref, b3_ref, acc_ref):
    j = pl.program_id(0)
    x = x_ref[...]
    h = jnp.maximum(_mmd(x, w1_ref[...]) + b1_ref[...], 0.0)
    h = jnp.maximum(_mmd(h, w2_ref[...]) + b2_ref[...], 0.0)
    hv = _mmd(h, w3_ref[...]) + b3_ref[...]
    bat = bat_ref[...]
    ohf = (bat == lax.broadcasted_iota(jnp.int32, (128, _B), 1)).astype(_f32)
    gm = jnp.sum(ohf * gmax_ref[0:1, :], axis=1, keepdims=True)
    valid = bat < _B
    ge = jnp.where(valid, jnp.exp(gate_ref[...] - gm), 0.0)
    ohge = ohf * ge
    hv1 = jnp.concatenate([hv, jnp.ones_like(hv)], axis=1)
    contrib = lax.dot_general(ohge, hv1, (((0,), (0,)), ((), ())),
                              preferred_element_type=_f32,
                              precision=lax.Precision.HIGHEST)

    @pl.when(j == 0)
    def _():
        acc_ref[...] = jnp.zeros((_B, 2), _f32)

    acc_ref[...] += contrib


def _pass2(x, gate, gmax, batch2d, W1, b1, W2, b2, W3, b3):
    return pl.pallas_call(
        _pass2_body,
        grid=(_NT,),
        in_specs=[
            pl.BlockSpec((128, _D), lambda j: (j, 0)),
            pl.BlockSpec((128, 1), lambda j: (j, 0)),
            pl.BlockSpec((8, _B), lambda j: (0, 0)),
            pl.BlockSpec((128, 1), lambda j: (j, 0)),
            pl.BlockSpec((_D, 128), lambda j: (0, 0)),
            pl.BlockSpec((1, 128), lambda j: (0, 0)),
            pl.BlockSpec((128, 128), lambda j: (0, 0)),
            pl.BlockSpec((1, 128), lambda j: (0, 0)),
            pl.BlockSpec((128, 1), lambda j: (0, 0)),
            pl.BlockSpec((1, 1), lambda j: (0, 0)),
        ],
        out_specs=pl.BlockSpec((_B, 2), lambda j: (0, 0)),
        out_shape=jax.ShapeDtypeStruct((_B, 2), _f32),
    )(x, gate, gmax, batch2d, W1, b1, W2, b2, W3, b3)


def _norm_body(acc_ref, o_ref):
    num = acc_ref[:, 0:1]
    den = acc_ref[:, 1:2]
    o_ref[...] = jnp.where(den > 0.0, num / den, 0.0)


def _norm(acc):
    return pl.pallas_call(
        _norm_body,
        out_shape=jax.ShapeDtypeStruct((_B, 1), _f32),
    )(acc)


def _sc_edge(xs, xdu, eterm, src2d, dst2d, zeros):
    mesh = plsc.VectorSubcoreMesh(core_axis_name="c", subcore_axis_name="s")

    @functools.partial(
        pl.kernel,
        out_type=[
            jax.ShapeDtypeStruct((2, 2, _NH, _D), _f32),
            jax.ShapeDtypeStruct((_EP, _D), _f32),
        ],
        mesh=mesh,
        scratch_types=[
            pltpu.VMEM((_CHUNKS, 128), jnp.int32),
            pltpu.VMEM((_CHUNKS, 128), jnp.int32),
            pltpu.VMEM((1, 128), jnp.int32),
            pltpu.VMEM((128, _D), _f32),
            pltpu.VMEM((128, _D), _f32),
            pltpu.VMEM((128, _D), _f32),
            pltpu.VMEM_SHARED((_NAGG, _D), _f32),
            pltpu.SemaphoreType.DMA,
            pltpu.SemaphoreType.DMA,
            pltpu.SemaphoreType.DMA,
        ],
    )
    def k(xs_hbm, xdu_hbm, et_hbm, src_hbm, dst_hbm, z_hbm,
          out_hbm, spill_hbm,
          src_v, dst_v, idx_v, av, bv, ev, agg_sh, s1, s2, s3):
        cid = lax.axis_index("c")
        sid = lax.axis_index("s")
        wid = cid * 16 + sid
        base = wid * _CHUNKS
        pltpu.sync_copy(src_hbm.at[pl.ds(base, _CHUNKS)], src_v)
        pltpu.sync_copy(dst_hbm.at[pl.ds(base, _CHUNKS)], dst_v)

        @pl.when(sid == 0)
        def _():
            pltpu.sync_copy(z_hbm, agg_sh)

        plsc.subcore_barrier()

        def chunk0(j, carry):
            ca = pltpu.async_copy(xs_hbm.at[src_v.at[j]], av, s1)
            cb = pltpu.async_copy(xdu_hbm.at[dst_v.at[j]], bv, s2)
            ce = pltpu.async_copy(
                et_hbm.at[pl.ds((base + j) * 128, 128)], ev, s3)
            ca.wait()
            cb.wait()
            ce.wait()

            def row(i, c2):
                for c in range(_D // 16):
                    sl = pl.ds(c * 16, 16)
                    av[i, sl] = jnp.maximum(
                        av[i, sl] + bv[i, sl] + ev[i, sl], 0.0)
                return c2

            lax.fori_loop(0, 128, row, 0)
            pltpu.sync_copy(av, spill_hbm.at[pl.ds((base + j) * 128, 128)])
            for c in range(128 // 16):
                sl = pl.ds(c * 16, 16)
                d = dst_v[j, sl]
                idx_v[0, sl] = jnp.minimum(d, _NH)
            pltpu.sync_copy(av, agg_sh.at[idx_v.at[0]], add=True)
            return carry

        lax.fori_loop(0, _CHUNKS, chunk0, 0)
        plsc.subcore_barrier()

        @pl.when(sid == 0)
        def _():
            pltpu.sync_copy(agg_sh.at[pl.ds(0, _NH)], out_hbm.at[cid, 0])
            pltpu.sync_copy(z_hbm, agg_sh)

        plsc.subcore_barrier()

        def chunk1(j, carry):
            pltpu.sync_copy(spill_hbm.at[pl.ds((base + j) * 128, 128)], av)
            for c in range(128 // 16):
                sl = pl.ds(c * 16, 16)
                d = dst_v[j, sl]
                idx_v[0, sl] = jnp.where(d < _NH, _NH, d - _NH)
            pltpu.sync_copy(av, agg_sh.at[idx_v.at[0]], add=True)
            return carry

        lax.fori_loop(0, _CHUNKS, chunk1, 0)
        plsc.subcore_barrier()

        @pl.when(sid == 0)
        def _():
            pltpu.sync_copy(agg_sh.at[pl.ds(0, _NH)], out_hbm.at[cid, 1])

    return k(xs, xdu, eterm, src2d, dst2d, zeros)[0]


def kernel(node_attr, edge_index, edge_slot, edge_attr, u, batch,
           node_lin_W, node_lin_b, conv_Wm, conv_bm, conv_slot, conv_Wu,
           conv_bu, gate_W1, gate_b1, gate_W2, gate_b2, gate_W3, gate_b3,
           nn_W1, nn_b1, nn_W2, nn_b2, nn_W3, nn_b3):
    src = edge_index[0]
    dst = edge_index[1]

    x_na = jnp.pad(node_attr, ((0, _NP - _N), (0, 0)))
    batch2d = jnp.pad(batch, (0, _NP - _N), constant_values=_B)[:, None]
    src2d = jnp.pad(src, (0, _EP - _E)).reshape(_EROWS, 128)
    dst2d = jnp.pad(dst, (0, _EP - _E), constant_values=_N).reshape(_EROWS, 128)
    ea_p = jnp.pad(edge_attr, ((0, _EP - _E), (0, 0)))
    slot2d = jnp.pad(edge_slot, (0, _EP - _E))[:, None]
    zeros = jnp.zeros((_NAGG, _D), _f32)

    x = _node_lin(x_na, node_lin_W, node_lin_b[None, :])
    W3_all = conv_Wm[:, 2 * _D:, :]
    bdst2d = jnp.pad(batch[dst], (0, _EP - _E), constant_values=_B)[:, None]
    eterm_all = _eterm(ea_p, slot2d, bdst2d, u, W3_all, conv_slot, conv_bm)

    def conv(i, xin, res):
        Wm = conv_Wm[i]
        xs, xdu = _pre(xin, Wm[:_D], Wm[_D:2 * _D])
        agg_out = _sc_edge(xs, xdu, eterm_all[i], src2d, dst2d, zeros)
        Wu = conv_Wu[i]
        return _post(xin, agg_out, Wu[:_D], Wu[_D:], conv_bu[i][None, :], res)

    h = conv(0, x, None)
    x1 = conv(1, h, x)
    h = conv(2, x1, None)
    x2 = conv(3, h, x1)

    gate, gmax = _pass1(x2, batch2d, gate_W1, gate_b1[None, :],
                        gate_W2, gate_b2[None, :], gate_W3, gate_b3[None, :])
    acc = _pass2(x2, gate, gmax, batch2d, nn_W1, nn_b1[None, :],
                 nn_W2, nn_b2[None, :], nn_W3, nn_b3[None, :])
    return _norm(acc)


# speedup vs baseline: 1.3736x; 1.3736x over previous
import functools

import jax
import jax.numpy as jnp
from jax import lax
from jax.experimental import pallas as pl
from jax.experimental.pallas import tpu as pltpu
from jax.experimental.pallas import tpu_sc as plsc

_N = 10000
_E = 160000
_D = 128
_DE = 16
_DU = 16
_B = 64
_NSLOT = 8
_NCONV = 4

_NP = 10112
_NT = _NP // 128
_EP = 163840
_EROWS = _EP // 128
_WORKERS = 32
_CHUNKS = _EROWS // _WORKERS
_NH = 5120
_NAGG = _NH + 128
_ETILE = 1024
_ET = _EP // _ETILE

_f32 = jnp.float32


def _mm(a, b):
    return lax.dot_general(a, b, (((1,), (0,)), ((), ())),
                           preferred_element_type=_f32,
                           precision=lax.Precision.HIGHEST)


def _mmd(a, b):
    return lax.dot_general(a, b, (((1,), (0,)), ((), ())),
                           preferred_element_type=_f32)


def _node_lin_body(x_ref, w_ref, b_ref, o_ref):
    o_ref[...] = _mmd(x_ref[...], w_ref[...]) + b_ref[...]


def _node_lin(x, W, b):
    return pl.pallas_call(
        _node_lin_body,
        grid=(_NT,),
        in_specs=[
            pl.BlockSpec((128, _D), lambda j: (j, 0)),
            pl.BlockSpec((_D, _D), lambda j: (0, 0)),
            pl.BlockSpec((1, _D), lambda j: (0, 0)),
        ],
        out_specs=pl.BlockSpec((128, _D), lambda j: (j, 0)),
        out_shape=jax.ShapeDtypeStruct((_NP, _D), _f32),
    )(x, W, b)


def _pre_body(x_ref, wa_ref, wb_ref, u_ref, wc_ref, bat_ref,
              xs_ref, xdu_ref):
    x = x_ref[...]
    xs_ref[...] = _mmd(x, wa_ref[...])
    oh = (bat_ref[...] == lax.broadcasted_iota(jnp.int32, (128, _B), 1))
    ubn = _mm(oh.astype(_f32), u_ref[...])
    xdu_ref[...] = _mmd(x, wb_ref[...]) + _mmd(ubn, wc_ref[...])


def _pre(x, Wa, Wb, u, Wc, batch2d):
    full = jax.ShapeDtypeStruct((_NP, _D), _f32)
    return pl.pallas_call(
        _pre_body,
        grid=(_NT,),
        in_specs=[
            pl.BlockSpec((128, _D), lambda j: (j, 0)),
            pl.BlockSpec((_D, _D), lambda j: (0, 0)),
            pl.BlockSpec((_D, _D), lambda j: (0, 0)),
            pl.BlockSpec((_B, _DU), lambda j: (0, 0)),
            pl.BlockSpec((_DU, _D), lambda j: (0, 0)),
            pl.BlockSpec((128, 1), lambda j: (j, 0)),
        ],
        out_specs=[pl.BlockSpec((128, _D), lambda j: (j, 0))] * 2,
        out_shape=[full, full],
    )(x, Wa, Wb, u, Wc, batch2d)


def _eterm_body(ea_ref, slot_ref, we_ref, st_ref, bm_ref, o_ref):
    ea = ea_ref[...]
    oh = (slot_ref[...] == lax.broadcasted_iota(jnp.int32, (_ETILE, _NSLOT), 1))
    et = _mmd(ea, we_ref[0]) + bm_ref[0]
    et = et + _mm(oh.astype(_f32), st_ref[0])
    o_ref[...] = et[None]


def _eterm(ea, slot2d, We_all, slot_tbl, bm_all):
    return pl.pallas_call(
        _eterm_body,
        grid=(_NCONV, _ET),
        in_specs=[
            pl.BlockSpec((_ETILE, _DE), lambda i, j: (j, 0)),
            pl.BlockSpec((_ETILE, 1), lambda i, j: (j, 0)),
            pl.BlockSpec((1, _DE, _D), lambda i, j: (i, 0, 0)),
            pl.BlockSpec((1, _NSLOT, _D), lambda i, j: (i, 0, 0)),
            pl.BlockSpec((1, 1, _D), lambda i, j: (i, 0, 0)),
        ],
        out_specs=pl.BlockSpec((1, _ETILE, _D), lambda i, j: (i, j, 0)),
        out_shape=jax.ShapeDtypeStruct((_NCONV, _EP, _D), _f32),
    )(ea, slot2d, We_all, slot_tbl, bm_all[:, None, :])


def _post_body(with_res, *refs):
    if with_res:
        (x_ref, a0_ref, a1_ref, wu1_ref, wu2_ref, bu_ref, res_ref,
         o_ref) = refs
    else:
        x_ref, a0_ref, a1_ref, wu1_ref, wu2_ref, bu_ref, o_ref = refs
    agg = a0_ref[0] + a1_ref[0]
    z = (_mmd(x_ref[...], wu1_ref[...]) + _mmd(agg, wu2_ref[...])
         + bu_ref[...])
    if with_res:
        z = z + res_ref[...]
    o_ref[...] = jnp.maximum(z, 0.0)


def _post(x, agg_out, Wu1, Wu2, bu, res=None):
    with_res = res is not None
    nt = _NH // 128
    in_specs = [
        pl.BlockSpec((128, _D), lambda j: (j, 0)),
        pl.BlockSpec((1, 128, _D), lambda j: (j // nt, j % nt, 0)),
        pl.BlockSpec((1, 128, _D), lambda j: (j // nt, j % nt, 0)),
        pl.BlockSpec((_D, _D), lambda j: (0, 0)),
        pl.BlockSpec((_D, _D), lambda j: (0, 0)),
        pl.BlockSpec((1, _D), lambda j: (0, 0)),
    ]
    args = [x, agg_out[0], agg_out[1], Wu1, Wu2, bu]
    if with_res:
        in_specs.append(pl.BlockSpec((128, _D), lambda j: (j, 0)))
        args.append(res)
    return pl.pallas_call(
        functools.partial(_post_body, with_res),
        grid=(_NT,),
        in_specs=in_specs,
        out_specs=pl.BlockSpec((128, _D), lambda j: (j, 0)),
        out_shape=jax.ShapeDtypeStruct((_NP, _D), _f32),
    )(*args)


def _pass1_body(x_ref, bat_ref, w1_ref, b1_ref, w2_ref, b2_ref, w3_ref, b3_ref,
                gate_ref, gmax_ref):
    j = pl.program_id(0)
    x = x_ref[...]
    h = jnp.maximum(_mmd(x, w1_ref[...]) + b1_ref[...], 0.0)
    h = jnp.maximum(_mmd(h, w2_ref[...]) + b2_ref[...], 0.0)
    g = _mmd(h, w3_ref[...]) + b3_ref[...]
    gate_ref[...] = g
    oh = (bat_ref[...] == lax.broadcasted_iota(jnp.int32, (128, _B), 1))
    contrib = jnp.where(oh, g, -1e30)
    cmax = jnp.max(contrib, axis=0, keepdims=True)

    @pl.when(j == 0)
    def _():
        gmax_ref[...] = jnp.full((8, _B), -1e30, _f32)

    gmax_ref[...] = jnp.maximum(gmax_ref[...], jnp.broadcast_to(cmax, (8, _B)))


def _pass1(x, batch2d, W1, b1, W2, b2, W3, b3):
    return pl.pallas_call(
        _pass1_body,
        grid=(_NT,),
        in_specs=[
            pl.BlockSpec((128, _D), lambda j: (j, 0)),
            pl.BlockSpec((128, 1), lambda j: (j, 0)),
            pl.BlockSpec((_D, 128), lambda j: (0, 0)),
            pl.BlockSpec((1, 128), lambda j: (0, 0)),
            pl.BlockSpec((128, 128), lambda j: (0, 0)),
            pl.BlockSpec((1, 128), lambda j: (0, 0)),
            pl.BlockSpec((128, 1), lambda j: (0, 0)),
            pl.BlockSpec((1, 1), lambda j: (0, 0)),
        ],
        out_specs=[
            pl.BlockSpec((128, 1), lambda j: (j, 0)),
            pl.BlockSpec((8, _B), lambda j: (0, 0)),
        ],
        out_shape=[
            jax.ShapeDtypeStruct((_NP, 1), _f32),
            jax.ShapeDtypeStruct((8, _B), _f32),
        ],
    )(x, batch2d, W1, b1, W2, b2, W3, b3)


def _pass2_body(x_ref, gate_ref, gmax_ref, bat_ref,
                w1_ref, b1_ref, w2_ref, b2_ref, w3_ref, b3_ref, acc_ref):
    j = pl.program_id(0)
    x = x_ref[...]
    h = jnp.maximum(_mmd(x, w1_ref[...]) + b1_ref[...], 0.0)
    h = jnp.maximum(_mmd(h, w2_ref[...]) + b2_ref[...], 0.0)
    hv = _mmd(h, w3_ref[...]) + b3_ref[...]
    bat = bat_ref[...]
    ohf = (bat == lax.broadcasted_iota(jnp.int32, (128, _B), 1)).astype(_f32)
    gm = jnp.sum(ohf * gmax_ref[0:1, :], axis=1, keepdims=True)
    valid = bat < _B
    ge = jnp.where(valid, jnp.exp(gate_ref[...] - gm), 0.0)
    ohge = ohf * ge
    hv1 = jnp.concatenate([hv, jnp.ones_like(hv)], axis=1)
    contrib = lax.dot_general(ohge, hv1, (((0,), (0,)), ((), ())),
                              preferred_element_type=_f32,
                              precision=lax.Precision.HIGHEST)

    @pl.when(j == 0)
    def _():
        acc_ref[...] = jnp.zeros((_B, 2), _f32)

    acc_ref[...] += contrib


def _pass2(x, gate, gmax, batch2d, W1, b1, W2, b2, W3, b3):
    return pl.pallas_call(
        _pass2_body,
        grid=(_NT,),
        in_specs=[
            pl.BlockSpec((128, _D), lambda j: (j, 0)),
            pl.BlockSpec((128, 1), lambda j: (j, 0)),
            pl.BlockSpec((8, _B), lambda j: (0, 0)),
            pl.BlockSpec((128, 1), lambda j: (j, 0)),
            pl.BlockSpec((_D, 128), lambda j: (0, 0)),
            pl.BlockSpec((1, 128), lambda j: (0, 0)),
            pl.BlockSpec((128, 128), lambda j: (0, 0)),
            pl.BlockSpec((1, 128), lambda j: (0, 0)),
            pl.BlockSpec((128, 1), lambda j: (0, 0)),
            pl.BlockSpec((1, 1), lambda j: (0, 0)),
        ],
        out_specs=pl.BlockSpec((_B, 2), lambda j: (0, 0)),
        out_shape=jax.ShapeDtypeStruct((_B, 2), _f32),
    )(x, gate, gmax, batch2d, W1, b1, W2, b2, W3, b3)


def _norm_body(acc_ref, o_ref):
    num = acc_ref[:, 0:1]
    den = acc_ref[:, 1:2]
    o_ref[...] = jnp.where(den > 0.0, num / den, 0.0)


def _norm(acc):
    return pl.pallas_call(
        _norm_body,
        out_shape=jax.ShapeDtypeStruct((_B, 1), _f32),
    )(acc)


def _sc_edge(xs, xdu, eterm, src2d, dst2d, zeros):
    mesh = plsc.VectorSubcoreMesh(core_axis_name="c", subcore_axis_name="s")

    @functools.partial(
        pl.kernel,
        out_type=[
            jax.ShapeDtypeStruct((2, 2, _NH, _D), _f32),
            jax.ShapeDtypeStruct((_EP, _D), _f32),
        ],
        mesh=mesh,
        scratch_types=[
            pltpu.VMEM((_CHUNKS, 128), jnp.int32),
            pltpu.VMEM((_CHUNKS, 128), jnp.int32),
            pltpu.VMEM((1, 128), jnp.int32),
            pltpu.VMEM((128, _D), _f32),
            pltpu.VMEM((128, _D), _f32),
            pltpu.VMEM((128, _D), _f32),
            pltpu.VMEM_SHARED((_NAGG, _D), _f32),
            pltpu.SemaphoreType.DMA,
            pltpu.SemaphoreType.DMA,
            pltpu.SemaphoreType.DMA,
        ],
    )
    def k(xs_hbm, xdu_hbm, et_hbm, src_hbm, dst_hbm, z_hbm,
          out_hbm, spill_hbm,
          src_v, dst_v, idx_v, av, bv, ev, agg_sh, s1, s2, s3):
        cid = lax.axis_index("c")
        sid = lax.axis_index("s")
        wid = cid * 16 + sid
        base = wid * _CHUNKS
        pltpu.sync_copy(src_hbm.at[pl.ds(base, _CHUNKS)], src_v)
        pltpu.sync_copy(dst_hbm.at[pl.ds(base, _CHUNKS)], dst_v)

        @pl.when(sid == 0)
        def _():
            pltpu.sync_copy(z_hbm, agg_sh)

        plsc.subcore_barrier()

        def chunk0(j, carry):
            ca = pltpu.async_copy(xs_hbm.at[src_v.at[j]], av, s1)
            cb = pltpu.async_copy(xdu_hbm.at[dst_v.at[j]], bv, s2)
            ce = pltpu.async_copy(
                et_hbm.at[pl.ds((base + j) * 128, 128)], ev, s3)
            ca.wait()
            cb.wait()
            ce.wait()

            def row(i, c2):
                for c in range(_D // 16):
                    sl = pl.ds(c * 16, 16)
                    av[i, sl] = jnp.maximum(
                        av[i, sl] + bv[i, sl] + ev[i, sl], 0.0)
                return c2

            lax.fori_loop(0, 128, row, 0)
            pltpu.sync_copy(av, spill_hbm.at[pl.ds((base + j) * 128, 128)])
            for c in range(128 // 16):
                sl = pl.ds(c * 16, 16)
                d = dst_v[j, sl]
                idx_v[0, sl] = jnp.minimum(d, _NH)
            pltpu.sync_copy(av, agg_sh.at[idx_v.at[0]], add=True)
            return carry

        lax.fori_loop(0, _CHUNKS, chunk0, 0)
        plsc.subcore_barrier()

        @pl.when(sid == 0)
        def _():
            pltpu.sync_copy(agg_sh.at[pl.ds(0, _NH)], out_hbm.at[cid, 0])
            pltpu.sync_copy(z_hbm, agg_sh)

        plsc.subcore_barrier()

        def chunk1(j, carry):
            pltpu.sync_copy(spill_hbm.at[pl.ds((base + j) * 128, 128)], av)
            for c in range(128 // 16):
                sl = pl.ds(c * 16, 16)
                d = dst_v[j, sl]
                idx_v[0, sl] = jnp.where(d < _NH, _NH, d - _NH)
            pltpu.sync_copy(av, agg_sh.at[idx_v.at[0]], add=True)
            return carry

        lax.fori_loop(0, _CHUNKS, chunk1, 0)
        plsc.subcore_barrier()

        @pl.when(sid == 0)
        def _():
            pltpu.sync_copy(agg_sh.at[pl.ds(0, _NH)], out_hbm.at[cid, 1])

    return k(xs, xdu, eterm, src2d, dst2d, zeros)[0]


def kernel(node_attr, edge_index, edge_slot, edge_attr, u, batch,
           node_lin_W, node_lin_b, conv_Wm, conv_bm, conv_slot, conv_Wu,
           conv_bu, gate_W1, gate_b1, gate_W2, gate_b2, gate_W3, gate_b3,
           nn_W1, nn_b1, nn_W2, nn_b2, nn_W3, nn_b3):
    src = edge_index[0]
    dst = edge_index[1]

    x_na = jnp.pad(node_attr, ((0, _NP - _N), (0, 0)))
    batch2d = jnp.pad(batch, (0, _NP - _N), constant_values=_B)[:, None]
    src2d = jnp.pad(src, (0, _EP - _E)).reshape(_EROWS, 128)
    dst2d = jnp.pad(dst, (0, _EP - _E), constant_values=_N).reshape(_EROWS, 128)
    ea_p = jnp.pad(edge_attr, ((0, _EP - _E), (0, 0)))
    slot2d = jnp.pad(edge_slot, (0, _EP - _E))[:, None]
    zeros = jnp.zeros((_NAGG, _D), _f32)

    x = _node_lin(x_na, node_lin_W, node_lin_b[None, :])
    We_all = conv_Wm[:, 2 * _D:2 * _D + _DE, :]
    eterm_all = _eterm(ea_p, slot2d, We_all, conv_slot, conv_bm)

    def conv(i, xin, res):
        Wm = conv_Wm[i]
        xs, xdu = _pre(xin, Wm[:_D], Wm[_D:2 * _D], u,
                       Wm[2 * _D + _DE:], batch2d)
        agg_out = _sc_edge(xs, xdu, eterm_all[i], src2d, dst2d, zeros)
        Wu = conv_Wu[i]
        return _post(xin, agg_out, Wu[:_D], Wu[_D:], conv_bu[i][None, :], res)

    h = conv(0, x, None)
    x1 = conv(1, h, x)
    h = conv(2, x1, None)
    x2 = conv(3, h, x1)

    gate, gmax = _pass1(x2, batch2d, gate_W1, gate_b1[None, :],
                        gate_W2, gate_b2[None, :], gate_W3, gate_b3[None, :])
    acc = _pass2(x2, gate, gmax, batch2d, nn_W1, nn_b1[None, :],
                 nn_W2, nn_b2[None, :], nn_W3, nn_b3[None, :])
    return _norm(acc)
